# baseline (speedup 1.0000x reference)
import functools

import jax
import jax.numpy as jnp
from jax.experimental import pallas as pl
from jax.experimental.pallas import tpu as pltpu


B, T, C_IN, C_OUT = 2, 2048, 2048, 512
T_TILE = 256
NT = T // T_TILE
PAD = 16
EXT = T_TILE + 2 * PAD


def _rapn_kernel(x_ref, wcat_ref, bc_ref, wl_ref, bl_ref, out_ref, xp_ref):
    t = pl.program_id(1)

    @pl.when(t == 0)
    def _zero_pad_rows():
        xp_ref[0:PAD, :] = jnp.zeros((PAD, C_IN), jnp.bfloat16)
        xp_ref[T + PAD:T + 2 * PAD, :] = jnp.zeros((PAD, C_IN), jnp.bfloat16)

    @pl.when(t < NT)
    def _cast_block():
        ws = pl.multiple_of(PAD + t * T_TILE, PAD)
        xp_ref[pl.ds(ws, T_TILE), :] = x_ref[0].astype(jnp.bfloat16)

    @pl.when(t > 0)
    def _compute_tile():
        s = pl.multiple_of((t - 1) * T_TILE, T_TILE)
        ext = xp_ref[pl.ds(s, EXT), :]
        ycat = jnp.dot(ext, wcat_ref[...], preferred_element_type=jnp.float32)
        y = (ycat[PAD - 1:PAD - 1 + T_TILE, 0:C_OUT]
             + ycat[PAD:PAD + T_TILE, C_OUT:2 * C_OUT]
             + ycat[PAD + 1:PAD + 1 + T_TILE, 2 * C_OUT:3 * C_OUT])
        y = jnp.maximum(y + bc_ref[...], 0.0)
        logits = jnp.dot(y, wl_ref[...], preferred_element_type=jnp.float32)
        p = jax.nn.sigmoid(logits + bl_ref[0, 0])
        out_ref[0, pl.ds(s, T_TILE), :] = p


@functools.partial(jax.jit, static_argnames=())
def _run(x, wcat, bc, wl, bl):
    out_t = pl.pallas_call(
        _rapn_kernel,
        grid=(B, NT + 1),
        in_specs=[
            pl.BlockSpec((1, T_TILE, C_IN),
                         lambda b, t: (b, jnp.minimum(t, NT - 1), 0)),
            pl.BlockSpec((C_IN, 3 * C_OUT), lambda b, t: (0, 0)),
            pl.BlockSpec((1, C_OUT), lambda b, t: (0, 0)),
            pl.BlockSpec((C_OUT, 1), lambda b, t: (0, 0)),
            pl.BlockSpec((1, 1), lambda b, t: (0, 0)),
        ],
        out_specs=pl.BlockSpec((1, T, 1), lambda b, t: (b, 0, 0)),
        out_shape=jax.ShapeDtypeStruct((B, T, 1), jnp.float32),
        scratch_shapes=[pltpu.VMEM((T + 2 * PAD, C_IN), jnp.bfloat16)],
        compiler_params=pltpu.CompilerParams(
            vmem_limit_bytes=64 * 1024 * 1024,
        ),
    )(x, wcat, bc, wl, bl)
    return out_t[:, :, 0]


def kernel(ref_nor, ref_abn, W_conv, b_conv, W_lin, b_lin, isTrain):
    del ref_abn, isTrain
    wcat = jnp.transpose(W_conv, (1, 2, 0)).reshape(
        C_IN, 3 * C_OUT).astype(jnp.bfloat16)
    bc = b_conv.reshape(1, C_OUT)
    wl = W_lin.reshape(C_OUT, 1).astype(jnp.float32)
    bl = b_lin.reshape(1, 1)
    return _run(ref_nor, wcat, bc, wl, bl)


# speedup vs baseline: 1.0013x; 1.0013x over previous
import functools

import jax
import jax.numpy as jnp
from jax.experimental import pallas as pl
from jax.experimental.pallas import tpu as pltpu


B, T, C_IN, C_OUT = 2, 2048, 2048, 512
T_TILE = 256
NT = T // T_TILE
PAD = 16
EXT = T_TILE + 2 * PAD


def _rapn_kernel(x_ref, wcat_ref, bc_ref, wl_ref, bl_ref, out_ref, xp_ref):
    t = pl.program_id(1)

    @pl.when(t == 0)
    def _zero_pad_rows():
        xp_ref[0:PAD, :] = jnp.zeros((PAD, C_IN), jnp.bfloat16)
        xp_ref[T + PAD:T + 2 * PAD, :] = jnp.zeros((PAD, C_IN), jnp.bfloat16)

    @pl.when(t < NT)
    def _cast_block():
        ws = pl.multiple_of(PAD + t * T_TILE, PAD)
        xp_ref[pl.ds(ws, T_TILE), :] = x_ref[0].astype(jnp.bfloat16)

    @pl.when(t > 0)
    def _compute_tile():
        s = pl.multiple_of((t - 1) * T_TILE, T_TILE)
        ext = xp_ref[pl.ds(s, EXT), :]
        ycat = jnp.dot(ext, wcat_ref[...], preferred_element_type=jnp.float32)
        y = (ycat[PAD - 1:PAD - 1 + T_TILE, 0:C_OUT]
             + ycat[PAD:PAD + T_TILE, C_OUT:2 * C_OUT]
             + ycat[PAD + 1:PAD + 1 + T_TILE, 2 * C_OUT:3 * C_OUT])
        y = jnp.maximum(y + bc_ref[...], 0.0)
        logits = jnp.dot(y, wl_ref[...], preferred_element_type=jnp.float32)
        p = jax.nn.sigmoid(logits + bl_ref[0, 0])
        out_ref[0, pl.ds(s, T_TILE), :] = p


@functools.partial(jax.jit, static_argnames=())
def _run(x, wcat, bc, wl, bl):
    out_t = pl.pallas_call(
        _rapn_kernel,
        grid=(B, NT + 1),
        in_specs=[
            pl.BlockSpec((1, T_TILE, C_IN),
                         lambda b, t: (b, jnp.minimum(t, NT - 1), 0)),
            pl.BlockSpec((C_IN, 3 * C_OUT), lambda b, t: (0, 0)),
            pl.BlockSpec((1, C_OUT), lambda b, t: (0, 0)),
            pl.BlockSpec((C_OUT, 1), lambda b, t: (0, 0)),
            pl.BlockSpec((1, 1), lambda b, t: (0, 0)),
        ],
        out_specs=pl.BlockSpec((1, T, 1), lambda b, t: (b, 0, 0)),
        out_shape=jax.ShapeDtypeStruct((B, T, 1), jnp.float32),
        scratch_shapes=[pltpu.VMEM((T + 2 * PAD, C_IN), jnp.bfloat16)],
        compiler_params=pltpu.CompilerParams(
            vmem_limit_bytes=64 * 1024 * 1024,
        ),
    )(x, wcat, bc, wl, bl)
    return out_t[:, :, 0]


def kernel(ref_nor, ref_abn, W_conv, b_conv, W_lin, b_lin, isTrain):
    del ref_abn, isTrain
    wcat = jnp.transpose(W_conv.astype(jnp.bfloat16), (1, 2, 0)).reshape(
        C_IN, 3 * C_OUT)
    bc = b_conv.reshape(1, C_OUT)
    wl = W_lin.reshape(C_OUT, 1).astype(jnp.float32)
    bl = b_lin.reshape(1, 1)
    return _run(ref_nor, wcat, bc, wl, bl)


# speedup vs baseline: 1.0550x; 1.0536x over previous
import functools

import jax
import jax.numpy as jnp
from jax.experimental import pallas as pl
from jax.experimental.pallas import tpu as pltpu


B, T, C_IN, C_OUT = 2, 2048, 2048, 512
T_TILE = 512
NT = T // T_TILE
PAD = 16
EXT = T_TILE + 2 * PAD


def _rapn_kernel(x_ref, wcat_ref, bc_ref, wl_ref, bl_ref, out_ref, xp_ref):
    t = pl.program_id(1)

    @pl.when(t == 0)
    def _zero_pad_rows():
        xp_ref[0:PAD, :] = jnp.zeros((PAD, C_IN), jnp.bfloat16)
        xp_ref[T + PAD:T + 2 * PAD, :] = jnp.zeros((PAD, C_IN), jnp.bfloat16)

    @pl.when(t < NT)
    def _cast_block():
        ws = pl.multiple_of(PAD + t * T_TILE, PAD)
        xp_ref[pl.ds(ws, T_TILE), :] = x_ref[0].astype(jnp.bfloat16)

    @pl.when(t > 0)
    def _compute_tile():
        s = pl.multiple_of((t - 1) * T_TILE, T_TILE)
        ext = xp_ref[pl.ds(s, EXT), :]
        ycat = jnp.dot(ext, wcat_ref[...], preferred_element_type=jnp.float32)
        y = (ycat[PAD - 1:PAD - 1 + T_TILE, 0:C_OUT]
             + ycat[PAD:PAD + T_TILE, C_OUT:2 * C_OUT]
             + ycat[PAD + 1:PAD + 1 + T_TILE, 2 * C_OUT:3 * C_OUT])
        y = jnp.maximum(y + bc_ref[...], 0.0)
        logits = jnp.dot(y, wl_ref[...], preferred_element_type=jnp.float32)
        p = jax.nn.sigmoid(logits + bl_ref[0, 0])
        out_ref[0, pl.ds(s, T_TILE), :] = p


@functools.partial(jax.jit, static_argnames=())
def _run(x, wcat, bc, wl, bl):
    out_t = pl.pallas_call(
        _rapn_kernel,
        grid=(B, NT + 1),
        in_specs=[
            pl.BlockSpec((1, T_TILE, C_IN),
                         lambda b, t: (b, jnp.minimum(t, NT - 1), 0)),
            pl.BlockSpec((C_IN, 3 * C_OUT), lambda b, t: (0, 0)),
            pl.BlockSpec((1, C_OUT), lambda b, t: (0, 0)),
            pl.BlockSpec((C_OUT, 1), lambda b, t: (0, 0)),
            pl.BlockSpec((1, 1), lambda b, t: (0, 0)),
        ],
        out_specs=pl.BlockSpec((1, T, 1), lambda b, t: (b, 0, 0)),
        out_shape=jax.ShapeDtypeStruct((B, T, 1), jnp.float32),
        scratch_shapes=[pltpu.VMEM((T + 2 * PAD, C_IN), jnp.bfloat16)],
        compiler_params=pltpu.CompilerParams(
            vmem_limit_bytes=64 * 1024 * 1024,
        ),
    )(x, wcat, bc, wl, bl)
    return out_t[:, :, 0]


def kernel(ref_nor, ref_abn, W_conv, b_conv, W_lin, b_lin, isTrain):
    del ref_abn, isTrain
    wcat = jnp.transpose(W_conv.astype(jnp.bfloat16), (1, 2, 0)).reshape(
        C_IN, 3 * C_OUT)
    bc = b_conv.reshape(1, C_OUT)
    wl = W_lin.reshape(C_OUT, 1).astype(jnp.float32)
    bl = b_lin.reshape(1, 1)
    return _run(ref_nor, wcat, bc, wl, bl)
